# baseline (speedup 1.0000x reference)
import functools

import jax
import jax.numpy as jnp
from jax import lax
from jax.experimental import pallas as pl
from jax.experimental.pallas import tpu as pltpu
from jax.experimental.pallas import tpu_sc as plsc

B = 16
N = 4096
MAX_INCLUDED = 512
SQ_CUT = 1.0
L = 16
CHUNKS = N // L
OUT_WORDS = MAX_INCLUDED * 3


def _sc_body(coords_hbm, ref_hbm, out_hbm, cbuf, rbuf, obuf):
    c = lax.axis_index("c")
    s = lax.axis_index("s")

    @pl.when(s < B // 2)
    def _():
        row = c * (B // 2) + s

        pltpu.sync_copy(coords_hbm.at[row], cbuf)
        pltpu.sync_copy(ref_hbm.at[row], rbuf)

        zeros_f = jnp.zeros((L,), jnp.float32)
        zeros_i = jnp.zeros((L,), jnp.int32)
        lane = lax.iota(jnp.int32, L)
        czero = lane >> 4

        def zbody(j, carry):
            obuf[pl.ds(j * L, L)] = zeros_f
            return carry
        lax.fori_loop(0, OUT_WORDS // L, zbody, 0)

        rx = rbuf[pl.ds(0, L)]
        ry = rbuf[pl.ds(L, L)]
        rz = rbuf[pl.ds(2 * L, L)]

        def body(i, off):
            p = lane + i * L
            x = plsc.load_gather(cbuf, [p, czero])
            y = plsc.load_gather(cbuf, [p, czero + 1])
            z = plsc.load_gather(cbuf, [p, czero + 2])
            dx = x - rx
            dy = y - ry
            dz = z - rz
            d2 = dx * dx + dy * dy + dz * dz
            m = d2 <= SQ_CUT
            pos = off + plsc.cumsum(m.astype(jnp.int32)) - 1
            valid = m & (pos < MAX_INCLUDED)
            fidx = pos * 3
            plsc.store_scatter(obuf, [fidx], dx, mask=valid)
            plsc.store_scatter(obuf, [fidx + 1], dy, mask=valid)
            plsc.store_scatter(obuf, [fidx + 2], dz, mask=valid)
            return off + plsc.all_reduce_population_count(m)

        lax.fori_loop(0, CHUNKS, body, zeros_i, unroll=8)

        pltpu.sync_copy(obuf, out_hbm.at[row])


@jax.jit
def _run(coords, ref_pad):
    mesh = plsc.VectorSubcoreMesh(core_axis_name="c", subcore_axis_name="s")
    k = functools.partial(
        pl.kernel,
        mesh=mesh,
        out_type=jax.ShapeDtypeStruct((B, OUT_WORDS), jnp.float32),
        compiler_params=pltpu.CompilerParams(
            needs_layout_passes=False,
            skip_device_barrier=True,
            use_tc_tiling_on_sc=False,
        ),
        scratch_types=[
            pltpu.VMEM((N, 3), jnp.float32),
            pltpu.VMEM((3 * L,), jnp.float32),
            pltpu.VMEM((OUT_WORDS,), jnp.float32),
        ],
    )(_sc_body)
    return k(coords, ref_pad)


def kernel(coords, ref):
    ref_pad = jnp.broadcast_to(ref[:, :, None], (B, 3, L)).reshape(B, 3 * L)
    out = _run(coords, ref_pad)
    return out.reshape(B, MAX_INCLUDED, 3)


# speedup vs baseline: 3.0049x; 3.0049x over previous
import functools

import jax
import jax.numpy as jnp
from jax import lax
from jax.experimental import pallas as pl
from jax.experimental.pallas import tpu as pltpu
from jax.experimental.pallas import tpu_sc as plsc

B = 16
N = 4096
MAX_INCLUDED = 512
SQ_CUT = 1.0
L = 16
CHUNKS = N // L
OUT_WORDS = MAX_INCLUDED * 3


def _sc_body(coords_hbm, ref_hbm, out_hbm, cbuf, rbuf, obuf):
    c = lax.axis_index("c")
    s = lax.axis_index("s")

    @pl.when(s < B // 2)
    def _():
        row = c * (B // 2) + s

        pltpu.sync_copy(coords_hbm.at[row], cbuf)
        pltpu.sync_copy(ref_hbm.at[row], rbuf)

        zeros_f = jnp.zeros((L,), jnp.float32)
        zeros_i = jnp.zeros((L,), jnp.int32)

        def zbody(j, carry):
            obuf[pl.ds(j * L, L)] = zeros_f
            return carry
        lax.fori_loop(0, OUT_WORDS // L, zbody, 0)

        rx = rbuf[pl.ds(0, L)]
        ry = rbuf[pl.ds(L, L)]
        rz = rbuf[pl.ds(2 * L, L)]

        def body(i, off):
            base = i * L
            x = cbuf[0, pl.ds(base, L)]
            y = cbuf[1, pl.ds(base, L)]
            z = cbuf[2, pl.ds(base, L)]
            dx = x - rx
            dy = y - ry
            dz = z - rz
            d2 = dx * dx + dy * dy + dz * dz
            m = d2 <= SQ_CUT
            pos = off + plsc.cumsum(m.astype(jnp.int32)) - 1
            valid = m & (pos < MAX_INCLUDED)
            fidx = pos * 3
            plsc.store_scatter(obuf, [fidx], dx, mask=valid)
            plsc.store_scatter(obuf, [fidx + 1], dy, mask=valid)
            plsc.store_scatter(obuf, [fidx + 2], dz, mask=valid)
            return off + plsc.all_reduce_population_count(m)

        lax.fori_loop(0, CHUNKS, body, zeros_i, unroll=8)

        pltpu.sync_copy(obuf, out_hbm.at[row])


@jax.jit
def _run(coords_t, ref_pad):
    mesh = plsc.VectorSubcoreMesh(core_axis_name="c", subcore_axis_name="s")
    k = functools.partial(
        pl.kernel,
        mesh=mesh,
        out_type=jax.ShapeDtypeStruct((B, OUT_WORDS), jnp.float32),
        compiler_params=pltpu.CompilerParams(
            needs_layout_passes=False,
            skip_device_barrier=True,
        ),
        scratch_types=[
            pltpu.VMEM((3, N), jnp.float32),
            pltpu.VMEM((3 * L,), jnp.float32),
            pltpu.VMEM((OUT_WORDS,), jnp.float32),
        ],
    )(_sc_body)
    return k(coords_t, ref_pad)


def kernel(coords, ref):
    coords_t = coords.transpose(0, 2, 1)
    ref_pad = jnp.broadcast_to(ref[:, :, None], (B, 3, L)).reshape(B, 3 * L)
    out = _run(coords_t, ref_pad)
    return out.reshape(B, MAX_INCLUDED, 3)


# speedup vs baseline: 3.1198x; 1.0382x over previous
import functools

import jax
import jax.numpy as jnp
from jax import lax
from jax.experimental import pallas as pl
from jax.experimental.pallas import tpu as pltpu
from jax.experimental.pallas import tpu_sc as plsc

B = 16
N = 4096
HALF = N // 2
MAX_INCLUDED = 512
SQ_CUT = 1.0
L = 16
HCHUNKS = HALF // L
OUT_WORDS = MAX_INCLUDED * 3


def _sc_body(coords_hbm, ref_hbm, out_hbm, cbuf, lbuf, nbuf, obuf, cntbuf,
             ncnt, shared_buf, shared_cnt):
    c = lax.axis_index("c")
    s = lax.axis_index("s")
    t = s // 2
    h = s % 2
    row = c * (B // 2) + t

    r3 = row * 3
    pltpu.sync_copy(coords_hbm.at[r3, pl.ds(h * HALF, HALF)],
                    cbuf.at[pl.ds(0, HALF)])
    pltpu.sync_copy(coords_hbm.at[r3 + 1, pl.ds(h * HALF, HALF)],
                    cbuf.at[pl.ds(HALF, HALF)])
    pltpu.sync_copy(coords_hbm.at[r3 + 2, pl.ds(h * HALF, HALF)],
                    cbuf.at[pl.ds(2 * HALF, HALF)])
    pltpu.sync_copy(ref_hbm.at[row], cntbuf)

    lane = lax.iota(jnp.int32, L)
    czero = lane >> 4
    zeros_i = jnp.zeros((L,), jnp.int32)

    rx = cntbuf[pl.ds(0, L)]
    ry = cntbuf[pl.ds(L, L)]
    rz = cntbuf[pl.ds(2 * L, L)]

    def body(i, off):
        base = i * L
        x = cbuf[pl.ds(base, L)]
        y = cbuf[pl.ds(HALF + base, L)]
        z = cbuf[pl.ds(2 * HALF + base, L)]
        dx = x - rx
        dy = y - ry
        dz = z - rz
        d2 = dx * dx + dy * dy + dz * dz
        m = d2 <= SQ_CUT
        pos = off + plsc.cumsum(m.astype(jnp.int32)) - 1
        valid = m & (pos < MAX_INCLUDED)
        plsc.store_scatter(lbuf, [pos], dx, mask=valid)
        plsc.store_scatter(lbuf, [pos + MAX_INCLUDED], dy, mask=valid)
        plsc.store_scatter(lbuf, [pos + 2 * MAX_INCLUDED], dz, mask=valid)
        return off + plsc.all_reduce_population_count(m)

    cnt = lax.fori_loop(0, HCHUNKS, body, zeros_i, unroll=8)

    ncnt[pl.ds(0, L)] = cnt
    pltpu.sync_copy(lbuf, shared_buf.at[pl.ds(s * OUT_WORDS, OUT_WORDS)])
    pltpu.sync_copy(ncnt, shared_cnt.at[pl.ds(s * L, L)])
    plsc.subcore_barrier()

    @pl.when(h == 0)
    def _():
        pltpu.sync_copy(shared_buf.at[pl.ds((s + 1) * OUT_WORDS, OUT_WORDS)], nbuf)
        pltpu.sync_copy(shared_cnt.at[pl.ds((s + 1) * L, L)], ncnt)
        c1 = ncnt[pl.ds(0, L)]
        total = cnt + c1

        def merge(j, carry):
            jvec = lane + j * L
            sel0 = jvec < cnt
            sel1 = (~sel0) & (jvec < total)
            idx1 = jnp.clip(jvec - cnt, 0, MAX_INCLUDED - 1)
            fbase = jvec * 3
            for k in range(3):
                v0 = plsc.load_gather(lbuf, [jvec + k * MAX_INCLUDED])
                v1 = plsc.load_gather(nbuf, [idx1 + k * MAX_INCLUDED])
                v = jnp.where(sel0, v0, jnp.where(sel1, v1, 0.0))
                plsc.store_scatter(obuf, [fbase + k], v)
            return carry

        lax.fori_loop(0, MAX_INCLUDED // L, merge, 0, unroll=4)
        pltpu.sync_copy(obuf, out_hbm.at[row])


@jax.jit
def _run(coords_t, ref_pad):
    mesh = plsc.VectorSubcoreMesh(core_axis_name="c", subcore_axis_name="s")
    k = functools.partial(
        pl.kernel,
        mesh=mesh,
        out_type=jax.ShapeDtypeStruct((B, OUT_WORDS), jnp.float32),
        compiler_params=pltpu.CompilerParams(
            needs_layout_passes=False,
            skip_device_barrier=True,
        ),
        scratch_types=[
            pltpu.VMEM((3 * HALF,), jnp.float32),
            pltpu.VMEM((OUT_WORDS,), jnp.float32),
            pltpu.VMEM((OUT_WORDS,), jnp.float32),
            pltpu.VMEM((OUT_WORDS,), jnp.float32),
            pltpu.VMEM((3 * L,), jnp.float32),
            pltpu.VMEM((L,), jnp.int32),
            pltpu.VMEM_SHARED((16 * OUT_WORDS,), jnp.float32),
            pltpu.VMEM_SHARED((16 * L,), jnp.int32),
        ],
    )(_sc_body)
    return k(coords_t, ref_pad)


def kernel(coords, ref):
    coords_t = coords.transpose(0, 2, 1).reshape(B * 3, N)
    ref_pad = jnp.broadcast_to(ref[:, :, None], (B, 3, L)).reshape(B, 3 * L)
    out = _run(coords_t, ref_pad)
    return out.reshape(B, MAX_INCLUDED, 3)


# speedup vs baseline: 3.1764x; 1.0182x over previous
import functools

import jax
import jax.numpy as jnp
from jax import lax
from jax.experimental import pallas as pl
from jax.experimental.pallas import tpu as pltpu
from jax.experimental.pallas import tpu_sc as plsc

B = 16
N = 4096
HALF = N // 2
MAX_INCLUDED = 512
SQ_CUT = 1.0
L = 16
HCHUNKS = HALF // L
OUT_WORDS = MAX_INCLUDED * 3
LSTR = HALF + L


def _sc_body(coords_hbm, ref_hbm, out_hbm, cbuf, lbuf, nbuf, obuf, cntbuf,
             ncnt, shared_buf, shared_cnt):
    c = lax.axis_index("c")
    s = lax.axis_index("s")
    t = s // 2
    h = s % 2
    row = c * (B // 2) + t

    r3 = row * 3
    pltpu.sync_copy(coords_hbm.at[r3, pl.ds(h * HALF, HALF)],
                    cbuf.at[pl.ds(0, HALF)])
    pltpu.sync_copy(coords_hbm.at[r3 + 1, pl.ds(h * HALF, HALF)],
                    cbuf.at[pl.ds(HALF, HALF)])
    pltpu.sync_copy(coords_hbm.at[r3 + 2, pl.ds(h * HALF, HALF)],
                    cbuf.at[pl.ds(2 * HALF, HALF)])
    pltpu.sync_copy(ref_hbm.at[row], cntbuf)

    lane = lax.iota(jnp.int32, L)
    czero = lane >> 4
    zeros_i = jnp.zeros((L,), jnp.int32)

    rx = cntbuf[pl.ds(0, L)]
    ry = cntbuf[pl.ds(L, L)]
    rz = cntbuf[pl.ds(2 * L, L)]

    def body(i, off):
        base = i * L
        x = cbuf[pl.ds(base, L)]
        y = cbuf[pl.ds(HALF + base, L)]
        z = cbuf[pl.ds(2 * HALF + base, L)]
        dx = x - rx
        dy = y - ry
        dz = z - rz
        d2 = dx * dx + dy * dy + dz * dz
        m = d2 <= SQ_CUT
        plsc.store_compressed(lbuf.at[pl.ds(off, L)], dx, mask=m)
        plsc.store_compressed(lbuf.at[pl.ds(LSTR + off, L)], dy, mask=m)
        plsc.store_compressed(lbuf.at[pl.ds(2 * LSTR + off, L)], dz, mask=m)
        return off + plsc.all_reduce_population_count(m)[0]

    cnt_s = lax.fori_loop(0, HCHUNKS, body, 0, unroll=8)
    cnt = zeros_i + cnt_s

    ncnt[pl.ds(0, L)] = cnt
    sb = s * OUT_WORDS
    pltpu.sync_copy(lbuf.at[pl.ds(0, MAX_INCLUDED)],
                    shared_buf.at[pl.ds(sb, MAX_INCLUDED)])
    pltpu.sync_copy(lbuf.at[pl.ds(LSTR, MAX_INCLUDED)],
                    shared_buf.at[pl.ds(sb + MAX_INCLUDED, MAX_INCLUDED)])
    pltpu.sync_copy(lbuf.at[pl.ds(2 * LSTR, MAX_INCLUDED)],
                    shared_buf.at[pl.ds(sb + 2 * MAX_INCLUDED, MAX_INCLUDED)])
    pltpu.sync_copy(ncnt, shared_cnt.at[pl.ds(s * L, L)])
    plsc.subcore_barrier()

    @pl.when(h == 0)
    def _():
        pltpu.sync_copy(shared_buf.at[pl.ds((s + 1) * OUT_WORDS, OUT_WORDS)], nbuf)
        pltpu.sync_copy(shared_cnt.at[pl.ds((s + 1) * L, L)], ncnt)
        c1 = ncnt[pl.ds(0, L)]
        total = cnt + c1

        def merge(j, carry):
            jvec = lane + j * L
            sel0 = jvec < cnt
            sel1 = (~sel0) & (jvec < total)
            idx1 = jnp.clip(jvec - cnt, 0, MAX_INCLUDED - 1)
            fbase = jvec * 3
            for k in range(3):
                v0 = plsc.load_gather(lbuf, [jvec + k * LSTR])
                v1 = plsc.load_gather(nbuf, [idx1 + k * MAX_INCLUDED])
                v = jnp.where(sel0, v0, jnp.where(sel1, v1, 0.0))
                plsc.store_scatter(obuf, [fbase + k], v)
            return carry

        lax.fori_loop(0, MAX_INCLUDED // L, merge, 0, unroll=4)
        pltpu.sync_copy(obuf, out_hbm.at[row])


@jax.jit
def _run(coords_t, ref_pad):
    mesh = plsc.VectorSubcoreMesh(core_axis_name="c", subcore_axis_name="s")
    k = functools.partial(
        pl.kernel,
        mesh=mesh,
        out_type=jax.ShapeDtypeStruct((B, OUT_WORDS), jnp.float32),
        compiler_params=pltpu.CompilerParams(
            needs_layout_passes=False,
            skip_device_barrier=True,
        ),
        scratch_types=[
            pltpu.VMEM((3 * HALF,), jnp.float32),
            pltpu.VMEM((3 * LSTR,), jnp.float32),
            pltpu.VMEM((OUT_WORDS,), jnp.float32),
            pltpu.VMEM((OUT_WORDS,), jnp.float32),
            pltpu.VMEM((3 * L,), jnp.float32),
            pltpu.VMEM((L,), jnp.int32),
            pltpu.VMEM_SHARED((16 * OUT_WORDS,), jnp.float32),
            pltpu.VMEM_SHARED((16 * L,), jnp.int32),
        ],
    )(_sc_body)
    return k(coords_t, ref_pad)


def kernel(coords, ref):
    coords_t = coords.transpose(0, 2, 1).reshape(B * 3, N)
    ref_pad = jnp.broadcast_to(ref[:, :, None], (B, 3, L)).reshape(B, 3 * L)
    out = _run(coords_t, ref_pad)
    return out.reshape(B, MAX_INCLUDED, 3)
